# baseline (speedup 1.0000x reference)
import functools

import jax
import jax.numpy as jnp
from jax import lax
from jax.experimental import pallas as pl
from jax.experimental.pallas import tpu as pltpu
from jax.experimental.pallas import tpu_sc as plsc

_N = 100000
_E = 6400000
_NROWS = 784
_NPAD = _NROWS * 128
_DUMMY = 100224
_TGT = _N - 1
_NC, _NS = 2, 16
_NW = _NC * _NS
_TROWS = _E // 128
_CR = 16
_TCH = _TROWS // _CR
_BASECH = _TCH // _NW
_EXTRACH = _TCH % _NW
_SLICE = _NPAD // _NS

_mesh = plsc.VectorSubcoreMesh(
    core_axis_name="c", subcore_axis_name="s", num_cores=_NC, num_subcores=_NS
)


def _init_const_bufs(ones_b, zeros_b):
    for i in range(8):
        ones_b[pl.ds(i * 16, 16)] = jnp.ones((16,), jnp.float32)
        zeros_b[pl.ds(i * 16, 16)] = jnp.zeros((16,), jnp.float32)


def _zero_slice(tbl, s, zeros_b):
    def zbody(i, carry):
        pltpu.sync_copy(zeros_b, tbl.at[pl.ds(s * _SLICE + i * 128, 128)])
        return carry

    lax.fori_loop(0, _SLICE // 128, zbody, 0)


def _zero_table(tab):
    z16 = jnp.zeros((16,), jnp.float32)

    def zb(i, cc):
        for kk in range(8):
            tab[pl.ds(i * 128 + kk * 16, 16)] = z16
        return cc

    lax.fori_loop(0, _NPAD // 128, zb, 0)


_H = _NPAD // 2


@functools.partial(
    pl.kernel,
    out_type=jax.ShapeDtypeStruct((_NW, _NPAD), jnp.float32),
    mesh=_mesh,
    compiler_params=pltpu.CompilerParams(needs_layout_passes=False),
    scratch_types=[
        pltpu.VMEM((_H,), jnp.float32),
        pltpu.VMEM((_H,), jnp.float32),
        pltpu.VMEM((_CR, 128), jnp.int32),
    ],
)
def _sc_deg(dst_hbm, deg_out, deg0, deg1, dstbuf):
    c = lax.axis_index("c")
    s = lax.axis_index("s")
    w = c * _NS + s
    z16 = jnp.zeros((16,), jnp.float32)

    def zb(i, cc):
        for kk in range(8):
            deg0[pl.ds(i * 128 + kk * 16, 16)] = z16
            deg1[pl.ds(i * 128 + kk * 16, 16)] = z16
        return cc

    lax.fori_loop(0, _H // 128, zb, 0)
    ones16 = jnp.ones((16,), jnp.float32)
    nch = _BASECH + (w < _EXTRACH).astype(jnp.int32)

    def chunk(i, carry):
        row0 = (w + _NW * i) * _CR
        pltpu.sync_copy(dst_hbm.at[pl.ds(row0, _CR)], dstbuf)

        def gb(j, cc):
            for kk in range(8):
                dv = dstbuf[j, pl.ds(kk * 16, 16)]
                m0 = dv < _H
                plsc.addupdate_scatter(deg0, [dv], ones16, mask=m0)
                dv1 = jnp.where(m0, 0, dv - _H)
                plsc.addupdate_scatter(deg1, [dv1], ones16, mask=jnp.logical_not(m0))
            return cc

        lax.fori_loop(0, _CR, gb, 0)
        return carry

    lax.fori_loop(0, nch, chunk, 0)
    pltpu.sync_copy(deg0, deg_out.at[w, pl.ds(0, _H)])
    pltpu.sync_copy(deg1, deg_out.at[w, pl.ds(_H, _H)])


@functools.partial(
    pl.kernel,
    out_type=(
        jax.ShapeDtypeStruct((_NC, _NPAD), jnp.float32),
        jax.ShapeDtypeStruct((_NC, _NPAD), jnp.float32),
    ),
    mesh=_mesh,
    compiler_params=pltpu.CompilerParams(needs_layout_passes=False),
    scratch_types=[
        pltpu.VMEM((_NPAD,), jnp.float32),
        pltpu.VMEM((_CR, 128), jnp.int32),
        pltpu.VMEM((_CR, 128), jnp.int32),
        pltpu.VMEM((_CR, 128), jnp.int32),
        pltpu.VMEM((_CR, 128), jnp.int32),
        pltpu.VMEM((_CR, 128), jnp.float32),
        pltpu.VMEM((_CR, 128), jnp.float32),
        pltpu.VMEM((128,), jnp.float32),
        pltpu.VMEM((128,), jnp.float32),
        pltpu.VMEM_SHARED((_NPAD,), jnp.float32),
        pltpu.VMEM_SHARED((_NPAD,), jnp.float32),
        pltpu.SemaphoreType.DMA,
        pltpu.SemaphoreType.DMA,
    ],
)
def _sc_scatter(
    src_hbm, dst_hbm, y_hbm, a_out, c2_out,
    ytab, srcbuf, dstbuf0, dstbuf1, midxbuf, valsbuf0, valsbuf1,
    ones_b, zeros_b, ash, c2sh, sem0, sem1,
):
    c = lax.axis_index("c")
    s = lax.axis_index("s")
    w = c * _NS + s
    _init_const_bufs(ones_b, zeros_b)
    _zero_slice(ash, s, zeros_b)
    _zero_slice(c2sh, s, zeros_b)
    pltpu.sync_copy(y_hbm, ytab)
    plsc.subcore_barrier()

    sets = ((dstbuf0, valsbuf0, sem0), (dstbuf1, valsbuf1, sem1))
    nch = _BASECH + (w < _EXTRACH).astype(jnp.int32)

    def _drain(p):
        dbuf, vbuf, sem = sets[p]

        def drain(j, cc):
            pltpu.make_async_copy(vbuf.at[j], ash.at[dbuf.at[j]], sem).wait()
            return cc

        lax.fori_loop(0, _CR, drain, 0)

    def _sub(i, p):
        dbuf, vbuf, sem = sets[p]

        @pl.when(i >= 2)
        def _():
            _drain(p)

        row0 = (w + _NW * i) * _CR
        pltpu.sync_copy(src_hbm.at[pl.ds(row0, _CR)], srcbuf)
        pltpu.sync_copy(dst_hbm.at[pl.ds(row0, _CR)], dbuf)

        def gbody(j, hits):
            for kk in range(8):
                k = kk * 16
                sv = srcbuf[j, pl.ds(k, 16)]
                dv = dbuf[j, pl.ds(k, 16)]
                vbuf[j, pl.ds(k, 16)] = plsc.load_gather(ytab, [sv])
                m = dv == _TGT
                midxbuf[j, pl.ds(k, 16)] = jnp.where(m, sv, _DUMMY)
                hits = hits + m.astype(jnp.int32)
            return hits

        hits = lax.fori_loop(0, _CR, gbody, jnp.zeros((16,), jnp.int32))

        def fire(j, cc):
            pltpu.async_copy(vbuf.at[j], ash.at[dbuf.at[j]], sem, add=True)
            return cc

        lax.fori_loop(0, _CR, fire, 0)

        nh = jnp.sum(hits)

        @pl.when(nh > 0)
        def _():
            def sbody(j, cc):
                pltpu.sync_copy(ones_b, c2sh.at[midxbuf.at[j]], add=True)
                return cc

            lax.fori_loop(0, _CR, sbody, 0)

    def chunk(i, carry):
        @pl.when(i % 2 == 0)
        def _():
            _sub(i, 0)

        @pl.when(i % 2 == 1)
        def _():
            _sub(i, 1)

        return carry

    lax.fori_loop(0, nch, chunk, 0)
    _drain(0)
    _drain(1)
    plsc.subcore_barrier()
    pltpu.sync_copy(
        ash.at[pl.ds(s * _SLICE, _SLICE)], a_out.at[c, pl.ds(s * _SLICE, _SLICE)]
    )
    pltpu.sync_copy(
        c2sh.at[pl.ds(s * _SLICE, _SLICE)], c2_out.at[c, pl.ds(s * _SLICE, _SLICE)]
    )


def _tc_prep_body(degp_ref, x_ref, dinv_ref, y_ref):
    d = jnp.full((_NROWS // 7, 128), 1.0, jnp.float32)
    for t in range(_NW):
        d = d + degp_ref[t]
    dinv = lax.rsqrt(d)
    dinv_ref[...] = dinv
    y_ref[...] = x_ref[...] * dinv


def _tc_final_body(ap_ref, c2p_ref, y_ref, dinv_ref, p_ref, out_ref):
    a = ap_ref[0:_NROWS, :] + ap_ref[_NROWS : 2 * _NROWS, :]
    c2 = c2p_ref[0:_NROWS, :] + c2p_ref[_NROWS : 2 * _NROWS, :]
    dinv = dinv_ref[...]
    y = y_ref[...]
    s1 = dinv * (a + y)
    row = lax.broadcasted_iota(jnp.int32, (_NROWS, 128), 0)
    col = lax.broadcasted_iota(jnp.int32, (_NROWS, 128), 1)
    v = row * 128 + col
    wv = jnp.where(v < _N, (c2 + (v == _TGT).astype(jnp.float32)) * dinv, 0.0)
    dinv_n = dinv_ref[_TGT // 128, _TGT % 128]
    t = []
    for f in range(16):
        w1f = p_ref[0, f]
        b1f = p_ref[1, f]
        t.append(jnp.sum(jnp.maximum(s1 * w1f + b1f, 0.0) * wv))
    outv = p_ref[5, 0]
    for g in range(8):
        zg = t[0] * p_ref[2, g]
        for f in range(1, 16):
            zg = zg + t[f] * p_ref[2, f * 8 + g]
        h2g = jnp.maximum(zg * dinv_n + p_ref[3, g], 0.0)
        outv = outv + h2g * p_ref[4, g]
    out_ref[...] = jnp.full((8, 128), outv, jnp.float32)


def kernel(x, edge_index, W1, b1, W2, b2, Wfc, bfc):
    src2d = edge_index[0].reshape(_TROWS, 128)
    dst2d = edge_index[1].reshape(_TROWS, 128)
    xp = jnp.pad(x[:, 0], (0, _NPAD - _N))

    deg_parts = _sc_deg(dst2d)

    _BR = _NROWS // 7
    dinv2d, y2d = pl.pallas_call(
        _tc_prep_body,
        grid=(7,),
        in_specs=[
            pl.BlockSpec((_NW, _BR, 128), lambda i: (0, i, 0)),
            pl.BlockSpec((_BR, 128), lambda i: (i, 0)),
        ],
        out_specs=[
            pl.BlockSpec((_BR, 128), lambda i: (i, 0)),
            pl.BlockSpec((_BR, 128), lambda i: (i, 0)),
        ],
        out_shape=(
            jax.ShapeDtypeStruct((_NROWS, 128), jnp.float32),
            jax.ShapeDtypeStruct((_NROWS, 128), jnp.float32),
        ),
    )(deg_parts.reshape(_NW, _NROWS, 128), xp.reshape(_NROWS, 128))

    a_parts, c2_parts = _sc_scatter(src2d, dst2d, y2d.reshape(_NPAD))

    params = jnp.zeros((8, 128), jnp.float32)
    params = params.at[0, :16].set(W1[0])
    params = params.at[1, :16].set(b1)
    params = params.at[2, :128].set(W2.reshape(-1))
    params = params.at[3, :8].set(b2)
    params = params.at[4, :8].set(Wfc[:, 0])
    params = params.at[5, 0].set(bfc[0])

    out8 = pl.pallas_call(
        _tc_final_body,
        out_shape=jax.ShapeDtypeStruct((8, 128), jnp.float32),
    )(
        a_parts.reshape(_NC * _NROWS, 128),
        c2_parts.reshape(_NC * _NROWS, 128),
        y2d,
        dinv2d,
        params,
    )
    return out8[0, 0:1]


# speedup vs baseline: 1.0563x; 1.0563x over previous
import functools

import jax
import jax.numpy as jnp
from jax import lax
from jax.experimental import pallas as pl
from jax.experimental.pallas import tpu as pltpu
from jax.experimental.pallas import tpu_sc as plsc

_N = 100000
_E = 6400000
_NROWS = 784
_NPAD = _NROWS * 128
_DUMMY = 100224
_TGT = _N - 1
_NC, _NS = 2, 16
_NW = _NC * _NS
_TROWS = _E // 128
_CR = 16
_TCH = _TROWS // _CR
_BASECH = _TCH // _NW
_EXTRACH = _TCH % _NW
_SLICE = _NPAD // _NS

_mesh = plsc.VectorSubcoreMesh(
    core_axis_name="c", subcore_axis_name="s", num_cores=_NC, num_subcores=_NS
)


def _init_const_bufs(ones_b, zeros_b):
    for i in range(8):
        ones_b[pl.ds(i * 16, 16)] = jnp.ones((16,), jnp.float32)
        zeros_b[pl.ds(i * 16, 16)] = jnp.zeros((16,), jnp.float32)


def _zero_slice(tbl, s, zeros_b):
    def zbody(i, carry):
        pltpu.sync_copy(zeros_b, tbl.at[pl.ds(s * _SLICE + i * 128, 128)])
        return carry

    lax.fori_loop(0, _SLICE // 128, zbody, 0)


def _zero_table(tab):
    z16 = jnp.zeros((16,), jnp.float32)

    def zb(i, cc):
        for kk in range(8):
            tab[pl.ds(i * 128 + kk * 16, 16)] = z16
        return cc

    lax.fori_loop(0, _NPAD // 128, zb, 0)


@functools.partial(
    pl.kernel,
    out_type=jax.ShapeDtypeStruct((_NW, _NPAD), jnp.float32),
    mesh=_mesh,
    compiler_params=pltpu.CompilerParams(needs_layout_passes=False),
    scratch_types=[
        pltpu.VMEM((_NPAD,), jnp.float32),
        pltpu.VMEM((_CR, 128), jnp.int32),
    ],
)
def _sc_deg(dst_hbm, deg_out, degtab, dstbuf):
    c = lax.axis_index("c")
    s = lax.axis_index("s")
    w = c * _NS + s
    _zero_table(degtab)
    ones16 = jnp.ones((16,), jnp.float32)
    nch = _BASECH + (w < _EXTRACH).astype(jnp.int32)

    def chunk(i, carry):
        row0 = (w + _NW * i) * _CR
        pltpu.sync_copy(dst_hbm.at[pl.ds(row0, _CR)], dstbuf)

        def gb(j, cc):
            for kk in range(8):
                dv = dstbuf[j, pl.ds(kk * 16, 16)]
                plsc.addupdate_scatter(degtab, [dv], ones16)
            return cc

        lax.fori_loop(0, _CR, gb, 0)
        return carry

    lax.fori_loop(0, nch, chunk, 0)
    pltpu.sync_copy(degtab, deg_out.at[w])


@functools.partial(
    pl.kernel,
    out_type=(
        jax.ShapeDtypeStruct((_NC, _NPAD), jnp.float32),
        jax.ShapeDtypeStruct((_NC, _NPAD), jnp.float32),
    ),
    mesh=_mesh,
    compiler_params=pltpu.CompilerParams(needs_layout_passes=False),
    scratch_types=[
        pltpu.VMEM((_NPAD,), jnp.float32),
        pltpu.VMEM((_CR, 128), jnp.int32),
        pltpu.VMEM((_CR, 128), jnp.int32),
        pltpu.VMEM((_CR, 128), jnp.int32),
        pltpu.VMEM((_CR, 128), jnp.int32),
        pltpu.VMEM((_CR, 128), jnp.float32),
        pltpu.VMEM((_CR, 128), jnp.float32),
        pltpu.VMEM((128,), jnp.float32),
        pltpu.VMEM((128,), jnp.float32),
        pltpu.VMEM_SHARED((_NPAD,), jnp.float32),
        pltpu.VMEM_SHARED((_NPAD,), jnp.float32),
        pltpu.SemaphoreType.DMA,
        pltpu.SemaphoreType.DMA,
    ],
)
def _sc_scatter(
    src_hbm, dst_hbm, y_hbm, a_out, c2_out,
    ytab, srcbuf, dstbuf0, dstbuf1, midxbuf, valsbuf0, valsbuf1,
    ones_b, zeros_b, ash, c2sh, sem0, sem1,
):
    c = lax.axis_index("c")
    s = lax.axis_index("s")
    w = c * _NS + s
    _init_const_bufs(ones_b, zeros_b)
    _zero_slice(ash, s, zeros_b)
    _zero_slice(c2sh, s, zeros_b)
    pltpu.sync_copy(y_hbm, ytab)
    plsc.subcore_barrier()

    sets = ((dstbuf0, valsbuf0, sem0), (dstbuf1, valsbuf1, sem1))
    nch = _BASECH + (w < _EXTRACH).astype(jnp.int32)

    def _drain(p):
        dbuf, vbuf, sem = sets[p]

        def drain(j, cc):
            pltpu.make_async_copy(vbuf.at[j], ash.at[dbuf.at[j]], sem).wait()
            return cc

        lax.fori_loop(0, _CR, drain, 0)

    def _sub(i, p):
        dbuf, vbuf, sem = sets[p]

        @pl.when(i >= 2)
        def _():
            _drain(p)

        row0 = (w + _NW * i) * _CR
        pltpu.sync_copy(src_hbm.at[pl.ds(row0, _CR)], srcbuf)
        pltpu.sync_copy(dst_hbm.at[pl.ds(row0, _CR)], dbuf)

        def gbody(j, hits):
            for kk in range(8):
                k = kk * 16
                sv = srcbuf[j, pl.ds(k, 16)]
                dv = dbuf[j, pl.ds(k, 16)]
                vbuf[j, pl.ds(k, 16)] = plsc.load_gather(ytab, [sv])
                m = dv == _TGT
                midxbuf[j, pl.ds(k, 16)] = jnp.where(m, sv, _DUMMY)
                hits = hits + m.astype(jnp.int32)
            return hits

        hits = lax.fori_loop(0, _CR, gbody, jnp.zeros((16,), jnp.int32))

        def fire(j, cc):
            pltpu.async_copy(vbuf.at[j], ash.at[dbuf.at[j]], sem, add=True)
            return cc

        lax.fori_loop(0, _CR, fire, 0)

        nh = jnp.sum(hits)

        @pl.when(nh > 0)
        def _():
            def sbody(j, cc):
                pltpu.sync_copy(ones_b, c2sh.at[midxbuf.at[j]], add=True)
                return cc

            lax.fori_loop(0, _CR, sbody, 0)

    def chunk(i, carry):
        @pl.when(i % 2 == 0)
        def _():
            _sub(i, 0)

        @pl.when(i % 2 == 1)
        def _():
            _sub(i, 1)

        return carry

    lax.fori_loop(0, nch, chunk, 0)
    _drain(0)
    _drain(1)
    plsc.subcore_barrier()
    pltpu.sync_copy(
        ash.at[pl.ds(s * _SLICE, _SLICE)], a_out.at[c, pl.ds(s * _SLICE, _SLICE)]
    )
    pltpu.sync_copy(
        c2sh.at[pl.ds(s * _SLICE, _SLICE)], c2_out.at[c, pl.ds(s * _SLICE, _SLICE)]
    )


def _tc_prep_body(degp_ref, x_ref, dinv_ref, y_ref):
    d = jnp.full((_NROWS // 7, 128), 1.0, jnp.float32)
    for t in range(_NW):
        d = d + degp_ref[t]
    dinv = lax.rsqrt(d)
    dinv_ref[...] = dinv
    y_ref[...] = x_ref[...] * dinv


def _tc_final_body(ap_ref, c2p_ref, y_ref, dinv_ref, p_ref, out_ref):
    a = ap_ref[0:_NROWS, :] + ap_ref[_NROWS : 2 * _NROWS, :]
    c2 = c2p_ref[0:_NROWS, :] + c2p_ref[_NROWS : 2 * _NROWS, :]
    dinv = dinv_ref[...]
    y = y_ref[...]
    s1 = dinv * (a + y)
    row = lax.broadcasted_iota(jnp.int32, (_NROWS, 128), 0)
    col = lax.broadcasted_iota(jnp.int32, (_NROWS, 128), 1)
    v = row * 128 + col
    wv = jnp.where(v < _N, (c2 + (v == _TGT).astype(jnp.float32)) * dinv, 0.0)
    dinv_n = dinv_ref[_TGT // 128, _TGT % 128]
    t = []
    for f in range(16):
        w1f = p_ref[0, f]
        b1f = p_ref[1, f]
        t.append(jnp.sum(jnp.maximum(s1 * w1f + b1f, 0.0) * wv))
    outv = p_ref[5, 0]
    for g in range(8):
        zg = t[0] * p_ref[2, g]
        for f in range(1, 16):
            zg = zg + t[f] * p_ref[2, f * 8 + g]
        h2g = jnp.maximum(zg * dinv_n + p_ref[3, g], 0.0)
        outv = outv + h2g * p_ref[4, g]
    out_ref[...] = jnp.full((8, 128), outv, jnp.float32)


def kernel(x, edge_index, W1, b1, W2, b2, Wfc, bfc):
    src2d = edge_index[0].reshape(_TROWS, 128)
    dst2d = edge_index[1].reshape(_TROWS, 128)
    xp = jnp.pad(x[:, 0], (0, _NPAD - _N))

    deg_parts = _sc_deg(dst2d)

    _BR = _NROWS // 7
    dinv2d, y2d = pl.pallas_call(
        _tc_prep_body,
        grid=(7,),
        in_specs=[
            pl.BlockSpec((_NW, _BR, 128), lambda i: (0, i, 0)),
            pl.BlockSpec((_BR, 128), lambda i: (i, 0)),
        ],
        out_specs=[
            pl.BlockSpec((_BR, 128), lambda i: (i, 0)),
            pl.BlockSpec((_BR, 128), lambda i: (i, 0)),
        ],
        out_shape=(
            jax.ShapeDtypeStruct((_NROWS, 128), jnp.float32),
            jax.ShapeDtypeStruct((_NROWS, 128), jnp.float32),
        ),
    )(deg_parts.reshape(_NW, _NROWS, 128), xp.reshape(_NROWS, 128))

    a_parts, c2_parts = _sc_scatter(src2d, dst2d, y2d.reshape(_NPAD))

    params = jnp.zeros((8, 128), jnp.float32)
    params = params.at[0, :16].set(W1[0])
    params = params.at[1, :16].set(b1)
    params = params.at[2, :128].set(W2.reshape(-1))
    params = params.at[3, :8].set(b2)
    params = params.at[4, :8].set(Wfc[:, 0])
    params = params.at[5, 0].set(bfc[0])

    out8 = pl.pallas_call(
        _tc_final_body,
        out_shape=jax.ShapeDtypeStruct((8, 128), jnp.float32),
    )(
        a_parts.reshape(_NC * _NROWS, 128),
        c2_parts.reshape(_NC * _NROWS, 128),
        y2d,
        dinv2d,
        params,
    )
    return out8[0, 0:1]


# speedup vs baseline: 1.2142x; 1.1494x over previous
import functools

import jax
import jax.numpy as jnp
from jax import lax
from jax.experimental import pallas as pl
from jax.experimental.pallas import tpu as pltpu
from jax.experimental.pallas import tpu_sc as plsc

_N = 100000
_E = 6400000
_NROWS = 784
_NPAD = _NROWS * 128
_DUMMY = 100224
_TGT = _N - 1
_NC, _NS = 2, 16
_NW = _NC * _NS
_TROWS = _E // 128
_CR = 16
_TCH = _TROWS // _CR
_BASECH = _TCH // _NW
_EXTRACH = _TCH % _NW
_SLICE = _NPAD // _NS

_mesh = plsc.VectorSubcoreMesh(
    core_axis_name="c", subcore_axis_name="s", num_cores=_NC, num_subcores=_NS
)


def _init_const_bufs(ones_b, zeros_b):
    for i in range(8):
        ones_b[pl.ds(i * 16, 16)] = jnp.ones((16,), jnp.float32)
        zeros_b[pl.ds(i * 16, 16)] = jnp.zeros((16,), jnp.float32)


def _zero_slice(tbl, s, zeros_b):
    def zbody(i, carry):
        pltpu.sync_copy(zeros_b, tbl.at[pl.ds(s * _SLICE + i * 128, 128)])
        return carry

    lax.fori_loop(0, _SLICE // 128, zbody, 0)


def _zero_table(tab):
    z16 = jnp.zeros((16,), jnp.float32)

    def zb(i, cc):
        for kk in range(8):
            tab[pl.ds(i * 128 + kk * 16, 16)] = z16
        return cc

    lax.fori_loop(0, _NPAD // 128, zb, 0)


@functools.partial(
    pl.kernel,
    out_type=jax.ShapeDtypeStruct((_NW, _NPAD), jnp.float32),
    mesh=_mesh,
    compiler_params=pltpu.CompilerParams(needs_layout_passes=False),
    scratch_types=[
        pltpu.VMEM((_NPAD,), jnp.float32),
        pltpu.VMEM((_CR, 128), jnp.int32),
        pltpu.VMEM((_CR, 128), jnp.int32),
        pltpu.SemaphoreType.DMA,
        pltpu.SemaphoreType.DMA,
    ],
)
def _sc_deg(dst_hbm, deg_out, degtab, dstbuf0, dstbuf1, sem0, sem1):
    c = lax.axis_index("c")
    s = lax.axis_index("s")
    w = c * _NS + s
    _zero_table(degtab)
    ones16 = jnp.ones((16,), jnp.float32)
    nch = _BASECH + (w < _EXTRACH).astype(jnp.int32)
    sets = ((dstbuf0, sem0), (dstbuf1, sem1))

    pltpu.async_copy(dst_hbm.at[pl.ds(w * _CR, _CR)], dstbuf0, sem0)
    pltpu.async_copy(dst_hbm.at[pl.ds((w + _NW) * _CR, _CR)], dstbuf1, sem1)

    def _sub(i, p):
        buf, sem = sets[p]
        row0 = (w + _NW * i) * _CR
        pltpu.make_async_copy(dst_hbm.at[pl.ds(row0, _CR)], buf, sem).wait()

        def gb(j, cc):
            for kk in range(8):
                dv = buf[j, pl.ds(kk * 16, 16)]
                plsc.addupdate_scatter(degtab, [dv], ones16)
            return cc

        lax.fori_loop(0, _CR, gb, 0)

        @pl.when(i + 2 < nch)
        def _():
            row2 = (w + _NW * (i + 2)) * _CR
            pltpu.async_copy(dst_hbm.at[pl.ds(row2, _CR)], buf, sem)

    def chunk(i, carry):
        @pl.when(i % 2 == 0)
        def _():
            _sub(i, 0)

        @pl.when(i % 2 == 1)
        def _():
            _sub(i, 1)

        return carry

    lax.fori_loop(0, nch, chunk, 0)
    pltpu.sync_copy(degtab, deg_out.at[w])


@functools.partial(
    pl.kernel,
    out_type=(
        jax.ShapeDtypeStruct((_NC, _NPAD), jnp.float32),
        jax.ShapeDtypeStruct((_NC, _NPAD), jnp.float32),
    ),
    mesh=_mesh,
    compiler_params=pltpu.CompilerParams(needs_layout_passes=False),
    scratch_types=[
        pltpu.VMEM((_NPAD,), jnp.float32),
        pltpu.VMEM((_CR, 128), jnp.int32),
        pltpu.VMEM((_CR, 128), jnp.int32),
        pltpu.VMEM((_CR, 128), jnp.int32),
        pltpu.VMEM((_CR, 128), jnp.int32),
        pltpu.VMEM((_CR, 128), jnp.float32),
        pltpu.VMEM((_CR, 128), jnp.float32),
        pltpu.VMEM((128,), jnp.float32),
        pltpu.VMEM((128,), jnp.float32),
        pltpu.VMEM_SHARED((_NPAD,), jnp.float32),
        pltpu.VMEM_SHARED((_NPAD,), jnp.float32),
        pltpu.SemaphoreType.DMA,
        pltpu.SemaphoreType.DMA,
    ],
)
def _sc_scatter(
    src_hbm, dst_hbm, y_hbm, a_out, c2_out,
    ytab, srcbuf, dstbuf0, dstbuf1, midxbuf, valsbuf0, valsbuf1,
    ones_b, zeros_b, ash, c2sh, sem0, sem1,
):
    c = lax.axis_index("c")
    s = lax.axis_index("s")
    w = c * _NS + s
    _init_const_bufs(ones_b, zeros_b)
    _zero_slice(ash, s, zeros_b)
    _zero_slice(c2sh, s, zeros_b)
    pltpu.sync_copy(y_hbm, ytab)
    plsc.subcore_barrier()

    sets = ((dstbuf0, valsbuf0, sem0), (dstbuf1, valsbuf1, sem1))
    nch = _BASECH + (w < _EXTRACH).astype(jnp.int32)

    def _drain(p):
        dbuf, vbuf, sem = sets[p]

        def drain(j, cc):
            pltpu.make_async_copy(vbuf.at[j], ash.at[dbuf.at[j]], sem).wait()
            return cc

        lax.fori_loop(0, _CR, drain, 0)

    def _sub(i, p):
        dbuf, vbuf, sem = sets[p]

        @pl.when(i >= 2)
        def _():
            _drain(p)

        row0 = (w + _NW * i) * _CR
        pltpu.sync_copy(src_hbm.at[pl.ds(row0, _CR)], srcbuf)
        pltpu.sync_copy(dst_hbm.at[pl.ds(row0, _CR)], dbuf)

        def gbody(j, hits):
            for kk in range(8):
                k = kk * 16
                sv = srcbuf[j, pl.ds(k, 16)]
                dv = dbuf[j, pl.ds(k, 16)]
                vbuf[j, pl.ds(k, 16)] = plsc.load_gather(ytab, [sv])
                m = dv == _TGT
                midxbuf[j, pl.ds(k, 16)] = jnp.where(m, sv, _DUMMY)
                hits = hits + m.astype(jnp.int32)
            return hits

        hits = lax.fori_loop(0, _CR, gbody, jnp.zeros((16,), jnp.int32))

        def fire(j, cc):
            pltpu.async_copy(vbuf.at[j], ash.at[dbuf.at[j]], sem, add=True)
            return cc

        lax.fori_loop(0, _CR, fire, 0)

        nh = jnp.sum(hits)

        @pl.when(nh > 0)
        def _():
            def sbody(j, cc):
                pltpu.sync_copy(ones_b, c2sh.at[midxbuf.at[j]], add=True)
                return cc

            lax.fori_loop(0, _CR, sbody, 0)

    def chunk(i, carry):
        @pl.when(i % 2 == 0)
        def _():
            _sub(i, 0)

        @pl.when(i % 2 == 1)
        def _():
            _sub(i, 1)

        return carry

    lax.fori_loop(0, nch, chunk, 0)
    _drain(0)
    _drain(1)
    plsc.subcore_barrier()
    pltpu.sync_copy(
        ash.at[pl.ds(s * _SLICE, _SLICE)], a_out.at[c, pl.ds(s * _SLICE, _SLICE)]
    )
    pltpu.sync_copy(
        c2sh.at[pl.ds(s * _SLICE, _SLICE)], c2_out.at[c, pl.ds(s * _SLICE, _SLICE)]
    )


def _tc_prep_body(degp_ref, x_ref, dinv_ref, y_ref):
    d = jnp.full((_NROWS // 7, 128), 1.0, jnp.float32)
    for t in range(_NW):
        d = d + degp_ref[t]
    dinv = lax.rsqrt(d)
    dinv_ref[...] = dinv
    y_ref[...] = x_ref[...] * dinv


def _tc_final_body(ap_ref, c2p_ref, y_ref, dinv_ref, p_ref, out_ref):
    a = ap_ref[0:_NROWS, :] + ap_ref[_NROWS : 2 * _NROWS, :]
    c2 = c2p_ref[0:_NROWS, :] + c2p_ref[_NROWS : 2 * _NROWS, :]
    dinv = dinv_ref[...]
    y = y_ref[...]
    s1 = dinv * (a + y)
    row = lax.broadcasted_iota(jnp.int32, (_NROWS, 128), 0)
    col = lax.broadcasted_iota(jnp.int32, (_NROWS, 128), 1)
    v = row * 128 + col
    wv = jnp.where(v < _N, (c2 + (v == _TGT).astype(jnp.float32)) * dinv, 0.0)
    dinv_n = dinv_ref[_TGT // 128, _TGT % 128]
    t = []
    for f in range(16):
        w1f = p_ref[0, f]
        b1f = p_ref[1, f]
        t.append(jnp.sum(jnp.maximum(s1 * w1f + b1f, 0.0) * wv))
    outv = p_ref[5, 0]
    for g in range(8):
        zg = t[0] * p_ref[2, g]
        for f in range(1, 16):
            zg = zg + t[f] * p_ref[2, f * 8 + g]
        h2g = jnp.maximum(zg * dinv_n + p_ref[3, g], 0.0)
        outv = outv + h2g * p_ref[4, g]
    out_ref[...] = jnp.full((8, 128), outv, jnp.float32)


def kernel(x, edge_index, W1, b1, W2, b2, Wfc, bfc):
    src2d = edge_index[0].reshape(_TROWS, 128)
    dst2d = edge_index[1].reshape(_TROWS, 128)
    xp = jnp.pad(x[:, 0], (0, _NPAD - _N))

    deg_parts = _sc_deg(dst2d)

    _BR = _NROWS // 7
    dinv2d, y2d = pl.pallas_call(
        _tc_prep_body,
        grid=(7,),
        in_specs=[
            pl.BlockSpec((_NW, _BR, 128), lambda i: (0, i, 0)),
            pl.BlockSpec((_BR, 128), lambda i: (i, 0)),
        ],
        out_specs=[
            pl.BlockSpec((_BR, 128), lambda i: (i, 0)),
            pl.BlockSpec((_BR, 128), lambda i: (i, 0)),
        ],
        out_shape=(
            jax.ShapeDtypeStruct((_NROWS, 128), jnp.float32),
            jax.ShapeDtypeStruct((_NROWS, 128), jnp.float32),
        ),
    )(deg_parts.reshape(_NW, _NROWS, 128), xp.reshape(_NROWS, 128))

    a_parts, c2_parts = _sc_scatter(src2d, dst2d, y2d.reshape(_NPAD))

    params = jnp.zeros((8, 128), jnp.float32)
    params = params.at[0, :16].set(W1[0])
    params = params.at[1, :16].set(b1)
    params = params.at[2, :128].set(W2.reshape(-1))
    params = params.at[3, :8].set(b2)
    params = params.at[4, :8].set(Wfc[:, 0])
    params = params.at[5, 0].set(bfc[0])

    out8 = pl.pallas_call(
        _tc_final_body,
        out_shape=jax.ShapeDtypeStruct((8, 128), jnp.float32),
    )(
        a_parts.reshape(_NC * _NROWS, 128),
        c2_parts.reshape(_NC * _NROWS, 128),
        y2d,
        dinv2d,
        params,
    )
    return out8[0, 0:1]


# speedup vs baseline: 1.3765x; 1.1337x over previous
import functools

import jax
import jax.numpy as jnp
from jax import lax
from jax.experimental import pallas as pl
from jax.experimental.pallas import tpu as pltpu
from jax.experimental.pallas import tpu_sc as plsc

_N = 100000
_E = 6400000
_NROWS = 784
_NPAD = _NROWS * 128
_DUMMY = 100224
_TGT = _N - 1
_NC, _NS = 2, 16
_NW = _NC * _NS
_TROWS = _E // 128
_CR = 16
_TCH = _TROWS // _CR
_BASECH = _TCH // _NW
_EXTRACH = _TCH % _NW
_SLICE = _NPAD // _NS

_mesh = plsc.VectorSubcoreMesh(
    core_axis_name="c", subcore_axis_name="s", num_cores=_NC, num_subcores=_NS
)


def _init_const_bufs(ones_b, zeros_b):
    for i in range(8):
        ones_b[pl.ds(i * 16, 16)] = jnp.ones((16,), jnp.float32)
        zeros_b[pl.ds(i * 16, 16)] = jnp.zeros((16,), jnp.float32)


def _zero_slice(tbl, s, zeros_b):
    def zbody(i, carry):
        pltpu.sync_copy(zeros_b, tbl.at[pl.ds(s * _SLICE + i * 128, 128)])
        return carry

    lax.fori_loop(0, _SLICE // 128, zbody, 0)


def _zero_table(tab):
    z16 = jnp.zeros((16,), jnp.float32)

    def zb(i, cc):
        for kk in range(8):
            tab[pl.ds(i * 128 + kk * 16, 16)] = z16
        return cc

    lax.fori_loop(0, _NPAD // 128, zb, 0)


@functools.partial(
    pl.kernel,
    out_type=jax.ShapeDtypeStruct((_NW, _NPAD), jnp.float32),
    mesh=_mesh,
    compiler_params=pltpu.CompilerParams(needs_layout_passes=False),
    scratch_types=[
        pltpu.VMEM((_NPAD,), jnp.float32),
        pltpu.VMEM((_CR, 128), jnp.int32),
        pltpu.VMEM((_CR, 128), jnp.int32),
        pltpu.SemaphoreType.DMA,
        pltpu.SemaphoreType.DMA,
    ],
)
def _sc_deg(dst_hbm, deg_out, degtab, dstbuf0, dstbuf1, sem0, sem1):
    c = lax.axis_index("c")
    s = lax.axis_index("s")
    w = c * _NS + s
    _zero_table(degtab)
    ones16 = jnp.ones((16,), jnp.float32)
    nch = _BASECH + (w < _EXTRACH).astype(jnp.int32)
    sets = ((dstbuf0, sem0), (dstbuf1, sem1))

    pltpu.async_copy(dst_hbm.at[pl.ds(w * _CR, _CR)], dstbuf0, sem0)
    pltpu.async_copy(dst_hbm.at[pl.ds((w + _NW) * _CR, _CR)], dstbuf1, sem1)

    def _sub(i, p):
        buf, sem = sets[p]
        row0 = (w + _NW * i) * _CR
        pltpu.make_async_copy(dst_hbm.at[pl.ds(row0, _CR)], buf, sem).wait()

        def gb(j, cc):
            for kk in range(8):
                dv = buf[j, pl.ds(kk * 16, 16)]
                plsc.addupdate_scatter(degtab, [dv], ones16)
            return cc

        lax.fori_loop(0, _CR, gb, 0)

        @pl.when(i + 2 < nch)
        def _():
            row2 = (w + _NW * (i + 2)) * _CR
            pltpu.async_copy(dst_hbm.at[pl.ds(row2, _CR)], buf, sem)

    def chunk(i, carry):
        @pl.when(i % 2 == 0)
        def _():
            _sub(i, 0)

        @pl.when(i % 2 == 1)
        def _():
            _sub(i, 1)

        return carry

    lax.fori_loop(0, nch, chunk, 0)
    pltpu.sync_copy(degtab, deg_out.at[w])


@functools.partial(
    pl.kernel,
    out_type=(
        jax.ShapeDtypeStruct((_NC, _NPAD), jnp.float32),
        jax.ShapeDtypeStruct((_NC, _NPAD), jnp.float32),
    ),
    mesh=_mesh,
    compiler_params=pltpu.CompilerParams(needs_layout_passes=False),
    scratch_types=[
        pltpu.VMEM((_NPAD,), jnp.float32),
        pltpu.VMEM((_CR, 128), jnp.int32),
        pltpu.VMEM((_CR, 128), jnp.int32),
        pltpu.VMEM((_CR, 128), jnp.int32),
        pltpu.VMEM((_CR, 128), jnp.int32),
        pltpu.VMEM((_CR, 128), jnp.float32),
        pltpu.VMEM((_CR, 128), jnp.float32),
        pltpu.VMEM((128,), jnp.float32),
        pltpu.VMEM((128,), jnp.float32),
        pltpu.VMEM_SHARED((_NPAD,), jnp.float32),
        pltpu.VMEM_SHARED((_NPAD,), jnp.float32),
        pltpu.SemaphoreType.DMA,
        pltpu.SemaphoreType.DMA,
        pltpu.SemaphoreType.DMA,
        pltpu.SemaphoreType.DMA,
    ],
)
def _sc_scatter(
    src_hbm, dst_hbm, y_hbm, a_out, c2_out,
    ytab, srcbuf, dstbuf0, dstbuf1, midxbuf, valsbuf0, valsbuf1,
    ones_b, zeros_b, ash, c2sh, sem0, sem1, sem_s, sem_d,
):
    c = lax.axis_index("c")
    s = lax.axis_index("s")
    w = c * _NS + s
    _init_const_bufs(ones_b, zeros_b)
    _zero_slice(ash, s, zeros_b)
    _zero_slice(c2sh, s, zeros_b)
    pltpu.sync_copy(y_hbm, ytab)
    plsc.subcore_barrier()

    sets = ((dstbuf0, valsbuf0, sem0), (dstbuf1, valsbuf1, sem1))
    nch = _BASECH + (w < _EXTRACH).astype(jnp.int32)

    def _drain(p):
        dbuf, vbuf, sem = sets[p]

        def drain(j, cc):
            pltpu.make_async_copy(vbuf.at[j], ash.at[dbuf.at[j]], sem).wait()
            return cc

        lax.fori_loop(0, _CR, drain, 0)

    def _sub(i, p):
        dbuf, vbuf, sem = sets[p]

        row0 = (w + _NW * i) * _CR
        pltpu.async_copy(src_hbm.at[pl.ds(row0, _CR)], srcbuf, sem_s)

        @pl.when(i >= 2)
        def _():
            _drain(p)

        pltpu.async_copy(dst_hbm.at[pl.ds(row0, _CR)], dbuf, sem_d)
        pltpu.make_async_copy(src_hbm.at[pl.ds(row0, _CR)], srcbuf, sem_s).wait()
        pltpu.make_async_copy(dst_hbm.at[pl.ds(row0, _CR)], dbuf, sem_d).wait()

        def gbody(j, hits):
            for kk in range(8):
                k = kk * 16
                sv = srcbuf[j, pl.ds(k, 16)]
                dv = dbuf[j, pl.ds(k, 16)]
                vbuf[j, pl.ds(k, 16)] = plsc.load_gather(ytab, [sv])
                m = dv == _TGT
                midxbuf[j, pl.ds(k, 16)] = jnp.where(m, sv, _DUMMY)
                hits = hits + m.astype(jnp.int32)
            return hits

        hits = lax.fori_loop(0, _CR, gbody, jnp.zeros((16,), jnp.int32))

        def fire(j, cc):
            pltpu.async_copy(vbuf.at[j], ash.at[dbuf.at[j]], sem, add=True)
            return cc

        lax.fori_loop(0, _CR, fire, 0)

        nh = jnp.sum(hits)

        @pl.when(nh > 0)
        def _():
            def sbody(j, cc):
                pltpu.sync_copy(ones_b, c2sh.at[midxbuf.at[j]], add=True)
                return cc

            lax.fori_loop(0, _CR, sbody, 0)

    def chunk(i, carry):
        @pl.when(i % 2 == 0)
        def _():
            _sub(i, 0)

        @pl.when(i % 2 == 1)
        def _():
            _sub(i, 1)

        return carry

    lax.fori_loop(0, nch, chunk, 0)
    _drain(0)
    _drain(1)
    plsc.subcore_barrier()
    pltpu.sync_copy(
        ash.at[pl.ds(s * _SLICE, _SLICE)], a_out.at[c, pl.ds(s * _SLICE, _SLICE)]
    )
    pltpu.sync_copy(
        c2sh.at[pl.ds(s * _SLICE, _SLICE)], c2_out.at[c, pl.ds(s * _SLICE, _SLICE)]
    )


def _tc_prep_body(degp_ref, x_ref, dinv_ref, y_ref):
    d = jnp.full((_NROWS // 7, 128), 1.0, jnp.float32)
    for t in range(_NW):
        d = d + degp_ref[t]
    dinv = lax.rsqrt(d)
    dinv_ref[...] = dinv
    y_ref[...] = x_ref[...] * dinv


def _tc_final_body(ap_ref, c2p_ref, y_ref, dinv_ref, p_ref, out_ref):
    a = ap_ref[0:_NROWS, :] + ap_ref[_NROWS : 2 * _NROWS, :]
    c2 = c2p_ref[0:_NROWS, :] + c2p_ref[_NROWS : 2 * _NROWS, :]
    dinv = dinv_ref[...]
    y = y_ref[...]
    s1 = dinv * (a + y)
    row = lax.broadcasted_iota(jnp.int32, (_NROWS, 128), 0)
    col = lax.broadcasted_iota(jnp.int32, (_NROWS, 128), 1)
    v = row * 128 + col
    wv = jnp.where(v < _N, (c2 + (v == _TGT).astype(jnp.float32)) * dinv, 0.0)
    dinv_n = dinv_ref[_TGT // 128, _TGT % 128]
    t = []
    for f in range(16):
        w1f = p_ref[0, f]
        b1f = p_ref[1, f]
        t.append(jnp.sum(jnp.maximum(s1 * w1f + b1f, 0.0) * wv))
    outv = p_ref[5, 0]
    for g in range(8):
        zg = t[0] * p_ref[2, g]
        for f in range(1, 16):
            zg = zg + t[f] * p_ref[2, f * 8 + g]
        h2g = jnp.maximum(zg * dinv_n + p_ref[3, g], 0.0)
        outv = outv + h2g * p_ref[4, g]
    out_ref[...] = jnp.full((8, 128), outv, jnp.float32)


def kernel(x, edge_index, W1, b1, W2, b2, Wfc, bfc):
    src2d = edge_index[0].reshape(_TROWS, 128)
    dst2d = edge_index[1].reshape(_TROWS, 128)
    xp = jnp.pad(x[:, 0], (0, _NPAD - _N))

    deg_parts = _sc_deg(dst2d)

    _BR = _NROWS // 7
    dinv2d, y2d = pl.pallas_call(
        _tc_prep_body,
        grid=(7,),
        in_specs=[
            pl.BlockSpec((_NW, _BR, 128), lambda i: (0, i, 0)),
            pl.BlockSpec((_BR, 128), lambda i: (i, 0)),
        ],
        out_specs=[
            pl.BlockSpec((_BR, 128), lambda i: (i, 0)),
            pl.BlockSpec((_BR, 128), lambda i: (i, 0)),
        ],
        out_shape=(
            jax.ShapeDtypeStruct((_NROWS, 128), jnp.float32),
            jax.ShapeDtypeStruct((_NROWS, 128), jnp.float32),
        ),
    )(deg_parts.reshape(_NW, _NROWS, 128), xp.reshape(_NROWS, 128))

    a_parts, c2_parts = _sc_scatter(src2d, dst2d, y2d.reshape(_NPAD))

    params = jnp.zeros((8, 128), jnp.float32)
    params = params.at[0, :16].set(W1[0])
    params = params.at[1, :16].set(b1)
    params = params.at[2, :128].set(W2.reshape(-1))
    params = params.at[3, :8].set(b2)
    params = params.at[4, :8].set(Wfc[:, 0])
    params = params.at[5, 0].set(bfc[0])

    out8 = pl.pallas_call(
        _tc_final_body,
        out_shape=jax.ShapeDtypeStruct((8, 128), jnp.float32),
    )(
        a_parts.reshape(_NC * _NROWS, 128),
        c2_parts.reshape(_NC * _NROWS, 128),
        y2d,
        dinv2d,
        params,
    )
    return out8[0, 0:1]
